# Initial kernel scaffold; baseline (speedup 1.0000x reference)
import jax
import jax.numpy as jnp
from jax.experimental import pallas as pl


def kernel(x, pe_table):
    raise NotImplementedError("write your pallas kernel here")


# speedup vs baseline: 3.4204x; 3.4204x over previous
import jax
import jax.numpy as jnp
from jax.experimental import pallas as pl


def kernel(x, pe_table):
    T = x.shape[1]
    D = pe_table.shape[1]
    BLOCK = 512

    def body(in_ref, out_ref):
        out_ref[...] = in_ref[...]

    out = pl.pallas_call(
        body,
        grid=(T // BLOCK,),
        in_specs=[pl.BlockSpec((BLOCK, D), lambda i: (i, 0))],
        out_specs=pl.BlockSpec((BLOCK, D), lambda i: (i, 0)),
        out_shape=jax.ShapeDtypeStruct((T, D), pe_table.dtype),
    )(pe_table)
    return out[None]
